# baseline (speedup 1.0000x reference)
import functools

import jax
from jax import lax
from jax.experimental import pallas as pl
from jax.experimental.pallas import tpu as pltpu
from jax.experimental.pallas import tpu_sc as plsc

_CHUNK_ROWS = 32
_TILE_CHUNKS = 5


def _make_sc_copy(seq_len, d_model, dtype):
    info = plsc.get_sparse_core_info()
    nc, ns = info.num_cores, info.num_subcores
    nw = nc * ns
    rows_per = seq_len // nw
    nchunks = rows_per // _CHUNK_ROWS
    n_tile = _TILE_CHUNKS
    n_sp = nchunks - n_tile
    mesh = plsc.VectorSubcoreMesh(core_axis_name="c", subcore_axis_name="s")

    scratch = [
        pltpu.VMEM((_CHUNK_ROWS, d_model), dtype),
        pltpu.VMEM((_CHUNK_ROWS, d_model), dtype),
        pltpu.VMEM_SHARED((2 * ns, _CHUNK_ROWS, d_model), dtype),
    ]
    scratch += [pltpu.SemaphoreType.DMA] * 8

    @functools.partial(
        pl.kernel,
        mesh=mesh,
        out_type=jax.ShapeDtypeStruct((seq_len, d_model), dtype),
        scratch_types=scratch,
    )
    def sc_copy(table_hbm, out_hbm, tb0, tb1, shared, *sems):
        tg = sems[0:2]
        ts = sems[2:4]
        sg = sems[4:6]
        ss = sems[6:8]
        sid = lax.axis_index("s")
        wid = lax.axis_index("c") * ns + sid
        base = wid * rows_per
        tbufs = (tb0, tb1)
        tscat = [None, None]
        sscat = [None, None]
        for k in range(max(n_tile, n_sp)):
            b = k % 2
            gt = gs = None
            if k < n_tile:
                lo_t = base + k * _CHUNK_ROWS
                if tscat[b] is not None:
                    tscat[b].wait()
                gt = pltpu.async_copy(
                    table_hbm.at[pl.ds(lo_t, _CHUNK_ROWS)], tbufs[b], tg[b]
                )
            if k < n_sp:
                lo_s = base + (n_tile + k) * _CHUNK_ROWS
                if sscat[b] is not None:
                    sscat[b].wait()
                gs = pltpu.async_copy(
                    table_hbm.at[pl.ds(lo_s, _CHUNK_ROWS)],
                    shared.at[2 * sid + b],
                    sg[b],
                )
            if gt is not None:
                gt.wait()
                tscat[b] = pltpu.async_copy(
                    tbufs[b], out_hbm.at[pl.ds(lo_t, _CHUNK_ROWS)], ts[b]
                )
            if gs is not None:
                gs.wait()
                sscat[b] = pltpu.async_copy(
                    shared.at[2 * sid + b],
                    out_hbm.at[pl.ds(lo_s, _CHUNK_ROWS)],
                    ss[b],
                )
        for b in (0, 1):
            if tscat[b] is not None:
                tscat[b].wait()
            if sscat[b] is not None:
                sscat[b].wait()

    return sc_copy


def kernel(x, pos_table):
    seq_len = x.shape[1]
    d_model = pos_table.shape[1]
    table = pos_table[:seq_len]
    out = _make_sc_copy(seq_len, d_model, pos_table.dtype)(table)
    return out[None]


# speedup vs baseline: 1.0151x; 1.0151x over previous
import functools

import jax
from jax import lax
from jax.experimental import pallas as pl
from jax.experimental.pallas import tpu as pltpu
from jax.experimental.pallas import tpu_sc as plsc

_CHUNK_ROWS = 32
_TILE_CHUNKS = 3


def _make_sc_copy(seq_len, d_model, dtype):
    info = plsc.get_sparse_core_info()
    nc, ns = info.num_cores, info.num_subcores
    nw = nc * ns
    rows_per = seq_len // nw
    nchunks = rows_per // _CHUNK_ROWS
    n_tile = _TILE_CHUNKS
    n_sp = nchunks - n_tile
    mesh = plsc.VectorSubcoreMesh(core_axis_name="c", subcore_axis_name="s")

    scratch = [
        pltpu.VMEM((_CHUNK_ROWS, d_model), dtype),
        pltpu.VMEM((_CHUNK_ROWS, d_model), dtype),
        pltpu.VMEM_SHARED((2 * ns, _CHUNK_ROWS, d_model), dtype),
    ]
    scratch += [pltpu.SemaphoreType.DMA] * 8

    @functools.partial(
        pl.kernel,
        mesh=mesh,
        out_type=jax.ShapeDtypeStruct((seq_len, d_model), dtype),
        scratch_types=scratch,
    )
    def sc_copy(table_hbm, out_hbm, tb0, tb1, shared, *sems):
        tg = sems[0:2]
        ts = sems[2:4]
        sg = sems[4:6]
        ss = sems[6:8]
        sid = lax.axis_index("s")
        wid = lax.axis_index("c") * ns + sid
        base = wid * rows_per
        tbufs = (tb0, tb1)
        tscat = [None, None]
        sscat = [None, None]
        for k in range(max(n_tile, n_sp)):
            b = k % 2
            gt = gs = None
            if k < n_tile:
                lo_t = base + k * _CHUNK_ROWS
                if tscat[b] is not None:
                    tscat[b].wait()
                gt = pltpu.async_copy(
                    table_hbm.at[pl.ds(lo_t, _CHUNK_ROWS)], tbufs[b], tg[b]
                )
            if k < n_sp:
                lo_s = base + (n_tile + k) * _CHUNK_ROWS
                if sscat[b] is not None:
                    sscat[b].wait()
                gs = pltpu.async_copy(
                    table_hbm.at[pl.ds(lo_s, _CHUNK_ROWS)],
                    shared.at[2 * sid + b],
                    sg[b],
                )
            if gt is not None:
                gt.wait()
                tscat[b] = pltpu.async_copy(
                    tbufs[b], out_hbm.at[pl.ds(lo_t, _CHUNK_ROWS)], ts[b]
                )
            if gs is not None:
                gs.wait()
                sscat[b] = pltpu.async_copy(
                    shared.at[2 * sid + b],
                    out_hbm.at[pl.ds(lo_s, _CHUNK_ROWS)],
                    ss[b],
                )
        for b in (0, 1):
            if tscat[b] is not None:
                tscat[b].wait()
            if sscat[b] is not None:
                sscat[b].wait()

    return sc_copy


def kernel(x, pos_table):
    seq_len = x.shape[1]
    d_model = pos_table.shape[1]
    table = pos_table[:seq_len]
    out = _make_sc_copy(seq_len, d_model, pos_table.dtype)(table)
    return out[None]
